# baseline (speedup 1.0000x reference)
import functools

import jax
import jax.numpy as jnp
from jax import lax
from jax.experimental import pallas as pl
from jax.experimental.pallas import tpu as pltpu
from jax.experimental.pallas import tpu_sc as plsc

_VOCAB = 1000000
_D = 64
_SEQ = 200
_SEQP = 256
_BATCH = 1024

_NC = 2
_NS = 16
_NW = _NC * _NS
_SEQ_PER_W = _BATCH // _NW


def _build_sc_call():
    mesh = plsc.VectorSubcoreMesh(core_axis_name="c", subcore_axis_name="s")

    @functools.partial(
        pl.kernel,
        mesh=mesh,
        compiler_params=pltpu.CompilerParams(use_tc_tiling_on_sc=False),
        out_type=jax.ShapeDtypeStruct((_BATCH * _SEQ, _D), jnp.float32),
        scratch_types=[
            pltpu.VMEM((_SEQ_PER_W, _SEQP), jnp.int32),
            pltpu.VMEM((_SEQ_PER_W, _SEQP), jnp.int32),
            pltpu.VMEM((2, _SEQP, _D), jnp.float32),
            pltpu.SemaphoreType.DMA,
            pltpu.SemaphoreType.DMA,
            pltpu.SemaphoreType.DMA,
            pltpu.SemaphoreType.DMA,
            pltpu.SemaphoreType.DMA,
        ],
    )
    def sc_gather(ids_hbm, remap_hbm, emb_hbm, wpe_hbm, out_hbm,
                  ids_v, dict_v, rows_v,
                  sem_r, sem_e0, sem_e1, sem_o0, sem_o1):
        wid = lax.axis_index("s") * _NC + lax.axis_index("c")
        seq0 = wid * _SEQ_PER_W
        sem_e = (sem_e0, sem_e1)
        sem_o = (sem_o0, sem_o1)

        pltpu.sync_copy(ids_hbm.at[pl.ds(seq0, _SEQ_PER_W)], ids_v)

        def remap_row(j):
            return pltpu.make_async_copy(
                remap_hbm.at[ids_v.at[j]], dict_v.at[j], sem_r)

        def fire_remap(j, carry):
            remap_row(j).start()
            return carry

        def drain_remap(j, carry):
            remap_row(j).wait()
            return carry

        lax.fori_loop(0, _SEQ_PER_W, fire_remap, 0)
        lax.fori_loop(0, _SEQ_PER_W, drain_remap, 0)

        def prefill(b):
            pltpu.sync_copy(wpe_hbm, rows_v.at[b])

        def start_emb(i, b):
            pltpu.make_async_copy(
                emb_hbm.at[dict_v.at[i]], rows_v.at[b], sem_e[b],
            ).start(add=True)

        def wait_emb(i, b):
            pltpu.make_async_copy(
                emb_hbm.at[dict_v.at[i]], rows_v.at[b], sem_e[b],
            ).wait()

        def out_copy(i, b):
            base = (seq0 + i) * _SEQ
            return pltpu.make_async_copy(
                rows_v.at[b].at[pl.ds(0, _SEQ)],
                out_hbm.at[pl.ds(base, _SEQ)],
                sem_o[b],
            )

        prefill(0)
        start_emb(0, 0)

        def step(i, b):
            wait_emb(i, b)

            @pl.when(i + 1 < _SEQ_PER_W)
            def _():
                @pl.when(i >= 1)
                def _():
                    out_copy(i - 1, 1 - b).wait()
                prefill(1 - b)
                start_emb(i + 1, 1 - b)

            out_copy(i, b).start()

        def pair(g, carry):
            step(2 * g, 0)
            step(2 * g + 1, 1)
            return carry

        lax.fori_loop(0, _SEQ_PER_W // 2, pair, 0)

        out_copy(_SEQ_PER_W - 2, 0).wait()
        out_copy(_SEQ_PER_W - 1, 1).wait()

    return sc_gather


_SC_CALL = _build_sc_call()


def kernel(input_ids, attention_mask, embedding_dict, input_ids2dict_ids, wpe):
    ids_p = jnp.pad(input_ids, ((0, 0), (0, _SEQP - _SEQ)))
    wpe_p = jnp.pad(wpe, ((0, _SEQP - _SEQ), (0, 0)))
    out2d = _SC_CALL(ids_p, input_ids2dict_ids, embedding_dict, wpe_p)
    return out2d.reshape(_BATCH, _SEQ, _D), attention_mask


# speedup vs baseline: 2.6616x; 2.6616x over previous
import functools

import jax
import jax.numpy as jnp
from jax import lax
from jax.experimental import pallas as pl
from jax.experimental.pallas import tpu as pltpu
from jax.experimental.pallas import tpu_sc as plsc

_VOCAB = 1000000
_D = 64
_DP = 128
_SEQ = 200
_BATCH = 1024

_NC = 2
_NS = 16
_NW = _NC * _NS
_SEQ_PER_W = _BATCH // _NW
_ROWS_PER_W = _SEQ_PER_W * _SEQ
_CSEQ = 2
_CROWS = _CSEQ * _SEQ
_NCHUNK = _SEQ_PER_W // _CSEQ


def _build_sc_call():
    mesh = plsc.VectorSubcoreMesh(core_axis_name="c", subcore_axis_name="s")

    @functools.partial(
        pl.kernel,
        mesh=mesh,
        compiler_params=pltpu.CompilerParams(use_tc_tiling_on_sc=False),
        out_type=jax.ShapeDtypeStruct((_BATCH * _SEQ, _D), jnp.float32),
        scratch_types=[
            pltpu.VMEM((_SEQ_PER_W, _SEQ), jnp.int32),
            pltpu.VMEM((_ROWS_PER_W,), jnp.int32),
            pltpu.VMEM((2, _CROWS, _DP), jnp.float32),
            pltpu.SemaphoreType.DMA,
            pltpu.SemaphoreType.DMA,
            pltpu.SemaphoreType.DMA,
            pltpu.SemaphoreType.DMA,
            pltpu.SemaphoreType.DMA,
        ],
    )
    def sc_gather(ids_hbm, remap_hbm, emb_hbm, wpe_hbm, out_hbm,
                  ids_v, dict_v, rows_v,
                  sem_r, sem_e0, sem_e1, sem_o0, sem_o1):
        wid = lax.axis_index("s") * _NC + lax.axis_index("c")
        seq0 = wid * _SEQ_PER_W
        row0 = wid * _ROWS_PER_W
        sem_e = (sem_e0, sem_e1)
        sem_o = (sem_o0, sem_o1)

        pltpu.sync_copy(ids_hbm.at[pl.ds(seq0, _SEQ_PER_W)], ids_v)

        def remap_row(j):
            return pltpu.make_async_copy(
                remap_hbm.at[ids_v.at[j]],
                dict_v.at[pl.ds(j * _SEQ, _SEQ)],
                sem_r,
            )

        def fire_remap(j, carry):
            remap_row(j).start()
            return carry

        def drain_remap(j, carry):
            remap_row(j).wait()
            return carry

        lax.fori_loop(0, _SEQ_PER_W, fire_remap, 0)
        lax.fori_loop(0, _SEQ_PER_W, drain_remap, 0)

        def dict_slice(i):
            return dict_v.at[pl.ds(i * _CROWS, _CROWS)]

        def prefill(b):
            pltpu.sync_copy(wpe_hbm, rows_v.at[b])

        def start_emb(i, b):
            pltpu.make_async_copy(
                emb_hbm.at[dict_slice(i)], rows_v.at[b], sem_e[b],
            ).start(add=True)

        def wait_emb(i, b):
            pltpu.make_async_copy(
                emb_hbm.at[dict_slice(i)], rows_v.at[b], sem_e[b],
            ).wait()

        def out_copy(i, b):
            base = row0 + i * _CROWS
            return pltpu.make_async_copy(
                rows_v.at[b].at[:, pl.ds(0, _D)],
                out_hbm.at[pl.ds(base, _CROWS)],
                sem_o[b],
            )

        prefill(0)
        start_emb(0, 0)

        def step(i, b):
            wait_emb(i, b)

            @pl.when(i + 1 < _NCHUNK)
            def _():
                @pl.when(i >= 1)
                def _():
                    out_copy(i - 1, 1 - b).wait()
                prefill(1 - b)
                start_emb(i + 1, 1 - b)

            out_copy(i, b).start()

        def pair(g, carry):
            step(2 * g, 0)
            step(2 * g + 1, 1)
            return carry

        lax.fori_loop(0, _NCHUNK // 2, pair, 0)

        out_copy(_NCHUNK - 2, 0).wait()
        out_copy(_NCHUNK - 1, 1).wait()

    return sc_gather


_SC_CALL = _build_sc_call()


def kernel(input_ids, attention_mask, embedding_dict, input_ids2dict_ids, wpe):
    emb_p = jnp.pad(embedding_dict, ((0, 0), (0, _DP - _D)))
    wpe2_p = jnp.pad(jnp.concatenate([wpe] * _CSEQ, axis=0),
                     ((0, 0), (0, _DP - _D)))
    out2d = _SC_CALL(input_ids, input_ids2dict_ids, emb_p, wpe2_p)
    return out2d.reshape(_BATCH, _SEQ, _D), attention_mask
